# Initial kernel scaffold; baseline (speedup 1.0000x reference)
import jax
import jax.numpy as jnp
from jax.experimental import pallas as pl


def kernel(coordinates, node_attrs, edge_index, W_embed, W_up, Wr1, Wr2, Wr3, W_int, W_mix, Wc, W_msg, W_skip):
    raise NotImplementedError("write your pallas kernel here")


# speedup vs baseline: 10.5859x; 10.5859x over previous
import functools

import jax
import jax.numpy as jnp
import numpy as np
from jax import lax
from jax.experimental import pallas as pl
from jax.experimental.pallas import tpu as pltpu
from jax.experimental.pallas import tpu_sc as plsc

R_MAX = 5.0
NUM_BESSEL = 8
MAX_ELL = 2
N_CHAN = 64
AVG_NEIGH = 16.0
SH_DIM = (MAX_ELL + 1) ** 2
LDX = tuple(int(l) for l in np.concatenate([np.full(2 * l + 1, l) for l in range(MAX_ELL + 1)]))

_NC = 2
_NS = 16
_NW = _NC * _NS
_CHUNK = 128

_FCH = 4
_FW = (N_CHAN * SH_DIM) // _FCH


def _sc_gather(table, idx):
    n_rows, d = table.shape
    m = idx.shape[0]
    nchunk = m // _CHUNK
    niter = (nchunk + _NW - 1) // _NW
    mesh = plsc.VectorSubcoreMesh(core_axis_name="c", subcore_axis_name="s",
                                  num_cores=_NC, num_subcores=_NS)

    @functools.partial(
        pl.kernel,
        out_type=jax.ShapeDtypeStruct((m, d), jnp.float32),
        mesh=mesh,
        scratch_types=[
            pltpu.VMEM((_CHUNK,), jnp.int32),
            pltpu.VMEM((_CHUNK, d), jnp.float32),
            pltpu.SemaphoreType.DMA,
        ],
        compiler_params=pltpu.CompilerParams(use_tc_tiling_on_sc=False),
    )
    def k(table_hbm, idx_hbm, out_hbm, idx_v, rows_v, sem):
        wid = lax.axis_index("s") * _NC + lax.axis_index("c")

        def body(t, carry):
            cid = wid + _NW * t

            @pl.when(cid < nchunk)
            def _():
                pltpu.sync_copy(idx_hbm.at[pl.ds(cid * _CHUNK, _CHUNK)], idx_v)
                pltpu.async_copy(table_hbm.at[idx_v], rows_v, sem).wait()
                pltpu.sync_copy(rows_v, out_hbm.at[pl.ds(cid * _CHUNK, _CHUNK)])

            return carry

        lax.fori_loop(0, niter, body, 0)

    return k(table, idx)


def _sc_scatter(msg, dst, zeros):
    e = dst.shape[0]
    n = zeros.shape[0]
    nchunk = e // _CHUNK
    niter = (nchunk + _NS - 1) // _NS
    rows_per_tile = n // _NS
    mesh = plsc.VectorSubcoreMesh(core_axis_name="c", subcore_axis_name="s",
                                  num_cores=_NC, num_subcores=_NS)

    @functools.partial(
        pl.kernel,
        out_type=jax.ShapeDtypeStruct((n, _FCH * _FW), jnp.float32),
        mesh=mesh,
        scratch_types=[
            pltpu.VMEM((_CHUNK,), jnp.int32),
            pltpu.VMEM((_CHUNK, _FW), jnp.float32),
            pltpu.VMEM_SHARED((n, _FW), jnp.float32),
        ],
        compiler_params=pltpu.CompilerParams(use_tc_tiling_on_sc=False),
    )
    def k(msg_hbm, dst_hbm, z_hbm, out_hbm, idx_v, rows_v, acc):
        c = lax.axis_index("c")
        s = lax.axis_index("s")
        rbase = s * rows_per_tile

        for fc in range(_FCH // _NC):
            colbase = (c * (_FCH // _NC) + fc) * _FW
            pltpu.sync_copy(z_hbm.at[pl.ds(rbase, rows_per_tile)],
                            acc.at[pl.ds(rbase, rows_per_tile)])
            plsc.subcore_barrier()

            def body(t, carry):
                cid = s + _NS * t

                @pl.when(cid < nchunk)
                def _():
                    pltpu.sync_copy(dst_hbm.at[pl.ds(cid * _CHUNK, _CHUNK)], idx_v)
                    pltpu.sync_copy(
                        msg_hbm.at[pl.ds(cid * _CHUNK, _CHUNK), pl.ds(colbase, _FW)],
                        rows_v)
                    pltpu.sync_copy(rows_v, acc.at[idx_v], add=True)

                return carry

            lax.fori_loop(0, niter, body, 0)
            plsc.subcore_barrier()
            pltpu.sync_copy(
                acc.at[pl.ds(rbase, rows_per_tile)],
                out_hbm.at[pl.ds(rbase, rows_per_tile), pl.ds(colbase, _FW)])
            plsc.subcore_barrier()

    return k(msg, dst, zeros)


def _tc_embed(na, w_embed, w_up0):
    n = na.shape[0]
    nb = 1000
    grid = n // nb

    def body(na_ref, we_ref, wu_ref, nf_ref, h_ref):
        nf = jnp.dot(na_ref[...], we_ref[...], preferred_element_type=jnp.float32)
        nf_ref[...] = nf
        h_ref[...] = jnp.dot(nf, wu_ref[...], preferred_element_type=jnp.float32)

    return pl.pallas_call(
        body,
        grid=(grid,),
        in_specs=[
            pl.BlockSpec((nb, na.shape[1]), lambda b: (b, 0)),
            pl.BlockSpec(w_embed.shape, lambda b: (0, 0)),
            pl.BlockSpec(w_up0.shape, lambda b: (0, 0)),
        ],
        out_specs=[
            pl.BlockSpec((nb, N_CHAN), lambda b: (b, 0)),
            pl.BlockSpec((nb, N_CHAN), lambda b: (b, 0)),
        ],
        out_shape=[
            jax.ShapeDtypeStruct((n, N_CHAN), jnp.float32),
            jax.ShapeDtypeStruct((n, N_CHAN), jnp.float32),
        ],
        compiler_params=pltpu.CompilerParams(dimension_semantics=("parallel",)),
    )(na, w_embed, w_up0)


def _tc_edge(csrc, cdst, hs, wr1, wr2, wr3r):
    e = hs.shape[0]
    be = 2000
    grid = e // be
    s3, s5, s15 = 3.0 ** 0.5, 5.0 ** 0.5, 15.0 ** 0.5

    def body(cs_ref, cd_ref, hs_ref, w1_ref, w2_ref, w3_ref, msg_ref):
        cs = cs_ref[...]
        cd = cd_ref[...]
        dx = cd[:, 0:1] - cs[:, 0:1]
        dy = cd[:, 1:2] - cs[:, 1:2]
        dz = cd[:, 2:3] - cs[:, 2:3]
        r = jnp.sqrt(dx * dx + dy * dy + dz * dz + 1e-9)
        inv_r = 1.0 / r
        x = dx * inv_r
        y = dy * inv_r
        z = dz * inv_r
        sh = [
            None,
            s3 * x, s3 * y, s3 * z,
            s15 * (x * y), s15 * (y * z), (s5 / 2.0) * (3.0 * z * z - 1.0),
            s15 * (x * z), (s15 / 2.0) * (x * x - y * y),
        ]
        nn = lax.broadcasted_iota(jnp.int32, (be, NUM_BESSEL), 1).astype(jnp.float32) + 1.0
        rb = ((2.0 / R_MAX) ** 0.5) * jnp.sin(nn * (jnp.pi / R_MAX) * r) / (r + 1e-9)
        xx = r * (1.0 / R_MAX)
        xx2 = xx * xx
        xx4 = xx2 * xx2
        xx6 = xx4 * xx2
        xx7 = xx6 * xx
        xx8 = xx7 * xx
        env = 1.0 - 28.0 * xx6 + 48.0 * xx7 - 21.0 * xx8
        env = jnp.where(xx < 1.0, env, 0.0)
        ef = rb * env
        rw = jnp.dot(ef, w1_ref[...], preferred_element_type=jnp.float32)
        rw = rw * lax.logistic(rw)
        rw = jnp.dot(rw, w2_ref[...], preferred_element_type=jnp.float32)
        rw = rw * lax.logistic(rw)
        wlm = jnp.dot(rw, w3_ref[...], preferred_element_type=jnp.float32)
        h = hs_ref[...]
        g = [wlm[:, l * N_CHAN:(l + 1) * N_CHAN] * h for l in range(MAX_ELL + 1)]
        pieces = [g[0]] + [g[LDX[m]] * sh[m] for m in range(1, SH_DIM)]
        msg_ref[...] = jnp.concatenate(pieces, axis=1)

    return pl.pallas_call(
        body,
        grid=(grid,),
        in_specs=[
            pl.BlockSpec((be, 16), lambda b: (b, 0)),
            pl.BlockSpec((be, 16), lambda b: (b, 0)),
            pl.BlockSpec((be, N_CHAN), lambda b: (b, 0)),
            pl.BlockSpec(wr1.shape, lambda b: (0, 0)),
            pl.BlockSpec(wr2.shape, lambda b: (0, 0)),
            pl.BlockSpec(wr3r.shape, lambda b: (0, 0)),
        ],
        out_specs=pl.BlockSpec((be, SH_DIM * N_CHAN), lambda b: (b, 0)),
        out_shape=jax.ShapeDtypeStruct((e, SH_DIM * N_CHAN), jnp.float32),
        compiler_params=pltpu.CompilerParams(dimension_semantics=("parallel",)),
    )(csrc, cdst, hs, wr1, wr2, wr3r)


def _tc_node(a, na, nf, w_int, w_mix, wck, w_msg, w_skip, w_up_next,
             has_mix, has_skip, emit_h):
    n = na.shape[0]
    nb = 1000
    grid = n // nb
    inv = 1.0 / AVG_NEIGH

    def body(a_ref, na_ref, nf_ref, wi_ref, wm_ref, wc_ref, wmsg_ref,
             wsk_ref, wun_ref, out_ref, h_ref):
        na_b = na_ref[...]
        b1 = None
        b2 = None
        for m in range(SH_DIM):
            am = a_ref[:, m * N_CHAN:(m + 1) * N_CHAN] * inv
            am = jnp.dot(am, wi_ref[LDX[m]], preferred_element_type=jnp.float32)
            if m == 0 and has_mix:
                am = am + jnp.dot(na_b, wm_ref[...], preferred_element_type=jnp.float32)
            if m == 0:
                b1 = am
                b2 = am * am
            else:
                b2 = b2 + am * am
        b3 = b2 * b1
        w0 = jnp.dot(na_b, wc_ref[0], preferred_element_type=jnp.float32)
        w1 = jnp.dot(na_b, wc_ref[1], preferred_element_type=jnp.float32)
        w2 = jnp.dot(na_b, wc_ref[2], preferred_element_type=jnp.float32)
        mm = w0 * b1 + w1 * b2 + w2 * b3
        out = jnp.dot(mm, wmsg_ref[...], preferred_element_type=jnp.float32)
        if has_skip:
            out = out + jnp.dot(nf_ref[...], wsk_ref[...],
                                preferred_element_type=jnp.float32)
        out_ref[...] = out
        if emit_h:
            h_ref[...] = jnp.dot(out, wun_ref[...], preferred_element_type=jnp.float32)
        else:
            h_ref[...] = out

    return pl.pallas_call(
        body,
        grid=(grid,),
        in_specs=[
            pl.BlockSpec((nb, SH_DIM * N_CHAN), lambda b: (b, 0)),
            pl.BlockSpec((nb, na.shape[1]), lambda b: (b, 0)),
            pl.BlockSpec((nb, N_CHAN), lambda b: (b, 0)),
            pl.BlockSpec(w_int.shape, lambda b: (0, 0, 0)),
            pl.BlockSpec(w_mix.shape, lambda b: (0, 0)),
            pl.BlockSpec(wck.shape, lambda b: (0, 0, 0)),
            pl.BlockSpec(w_msg.shape, lambda b: (0, 0)),
            pl.BlockSpec(w_skip.shape, lambda b: (0, 0)),
            pl.BlockSpec(w_up_next.shape, lambda b: (0, 0)),
        ],
        out_specs=[
            pl.BlockSpec((nb, N_CHAN), lambda b: (b, 0)),
            pl.BlockSpec((nb, N_CHAN), lambda b: (b, 0)),
        ],
        out_shape=[
            jax.ShapeDtypeStruct((n, N_CHAN), jnp.float32),
            jax.ShapeDtypeStruct((n, N_CHAN), jnp.float32),
        ],
        compiler_params=pltpu.CompilerParams(dimension_semantics=("parallel",)),
    )(a, na, nf, w_int, w_mix, wck, w_msg, w_skip, w_up_next)


def kernel(coordinates, node_attrs, edge_index, W_embed, W_up, Wr1, Wr2, Wr3,
           W_int, W_mix, Wc, W_msg, W_skip):
    n = coordinates.shape[0]
    e = edge_index.shape[1]
    src = edge_index[0].astype(jnp.int32)
    dst = edge_index[1].astype(jnp.int32)

    coords16 = jnp.pad(coordinates.astype(jnp.float32), ((0, 0), (0, 13)))
    idx_flat = jnp.concatenate([src, dst])
    cgath = _sc_gather(coords16, idx_flat)
    csrc = cgath[:e]
    cdst = cgath[e:]

    zeros = jnp.zeros((n, _FW), jnp.float32)

    wr3r = [Wr3[i].reshape(Wr3.shape[1], N_CHAN, MAX_ELL + 1)
            .transpose(0, 2, 1).reshape(Wr3.shape[1], (MAX_ELL + 1) * N_CHAN)
            for i in range(2)]
    wck = [Wc[i].reshape(Wc.shape[1], N_CHAN, 3).transpose(2, 0, 1)
           for i in range(2)]

    nf, h = _tc_embed(node_attrs, W_embed, W_up[0])
    outs = []
    for i in range(2):
        hs = _sc_gather(h, src)
        msg = _tc_edge(csrc, cdst, hs, Wr1[i], Wr2[i], wr3r[i])
        a = _sc_scatter(msg, dst, zeros)
        nf, h = _tc_node(a, node_attrs, nf, W_int[i], W_mix, wck[i],
                         W_msg[i], W_skip[i], W_up[1],
                         has_mix=(i == 0), has_skip=(i > 0), emit_h=(i == 0))
        outs.append(nf)
    return jnp.stack(outs, axis=0)
